# initial kernel scaffold (unmeasured)
import jax
import jax.numpy as jnp
from jax import lax
from jax.experimental import pallas as pl
from jax.experimental.pallas import tpu as pltpu

S = 2048
K = 4096
N = 8192
S_HALF = 1024
NCHUNK = 8
CN = N // NCHUNK


def kernel(O, Wo):
    O2 = O.reshape(S, K).astype(jnp.bfloat16)

    def body(o_ref, wo_hbm, out_ref,
             wo_vmem, send_buf, recv_buf,
             wo_sem, send_sem, recv_sem, credit_sem):
        my_x = lax.axis_index("x")
        my_y = lax.axis_index("y")
        peer = (1 - my_x, my_y)

        barrier = pltpu.get_barrier_semaphore()
        pl.semaphore_signal(barrier, inc=1, device_id=peer,
                            device_id_type=pl.DeviceIdType.MESH)
        pl.semaphore_wait(barrier, 1)

        own_start = my_x * S_HALF
        other_start = (1 - my_x) * S_HALF

        for c in range(NCHUNK):
            cp = pltpu.make_async_copy(
                wo_hbm.at[:, pl.ds(c * CN, CN)], wo_vmem, wo_sem)
            cp.start()
            cp.wait()
            wo_bf = wo_vmem[...].astype(jnp.bfloat16)

            other = jnp.dot(o_ref[pl.ds(other_start, S_HALF), :], wo_bf,
                            preferred_element_type=jnp.float32)
            send_buf[...] = other.astype(jnp.bfloat16)
            if c > 0:
                pl.semaphore_wait(credit_sem, 1)
            rdma = pltpu.make_async_remote_copy(
                src_ref=send_buf, dst_ref=recv_buf,
                send_sem=send_sem, recv_sem=recv_sem,
                device_id=peer, device_id_type=pl.DeviceIdType.MESH)
            rdma.start()

            own = jnp.dot(o_ref[pl.ds(own_start, S_HALF), :], wo_bf,
                          preferred_element_type=jnp.float32)
            rdma.wait()
            out_ref[0, :, pl.ds(c * CN, CN)] = (
                own + recv_buf[...].astype(jnp.float32))
            if c < NCHUNK - 1:
                pl.semaphore_signal(credit_sem, inc=1, device_id=peer,
                                    device_id_type=pl.DeviceIdType.MESH)

    return pl.pallas_call(
        body,
        out_shape=jax.ShapeDtypeStruct((1, S_HALF, N), jnp.float32),
        in_specs=[
            pl.BlockSpec(memory_space=pltpu.VMEM),
            pl.BlockSpec(memory_space=pltpu.ANY),
        ],
        out_specs=pl.BlockSpec(memory_space=pltpu.VMEM),
        scratch_shapes=[
            pltpu.VMEM((K, CN), jnp.float32),
            pltpu.VMEM((S_HALF, CN), jnp.bfloat16),
            pltpu.VMEM((S_HALF, CN), jnp.bfloat16),
            pltpu.SemaphoreType.DMA,
            pltpu.SemaphoreType.DMA,
            pltpu.SemaphoreType.DMA,
            pltpu.SemaphoreType.REGULAR,
        ],
        compiler_params=pltpu.CompilerParams(collective_id=0),
    )(O2, Wo)


# baseline (device time: 492116 ns/iter reference)
import jax
import jax.numpy as jnp
from jax import lax
from jax.experimental import pallas as pl
from jax.experimental.pallas import tpu as pltpu

S = 2048
K = 4096
N = 8192
S_HALF = 1024
NCHUNK = 16
CN = N // NCHUNK


def kernel(O, Wo):
    O2 = O.reshape(S, K).astype(jnp.bfloat16)

    def body(o_ref, wo_hbm, out_hbm,
             wo_vmem, send_buf, recv_buf, out_vmem,
             wo_sem, send_sem, recv_sem, out_sem, credit_sem):
        my_x = lax.axis_index("x")
        my_y = lax.axis_index("y")
        peer = (1 - my_x, my_y)

        barrier = pltpu.get_barrier_semaphore()
        pl.semaphore_signal(barrier, inc=1, device_id=peer,
                            device_id_type=pl.DeviceIdType.MESH)
        pl.semaphore_wait(barrier, 1)

        own_start = my_x * S_HALF
        other_start = (1 - my_x) * S_HALF

        def chunk_step(c, carry):
            cp = pltpu.make_async_copy(
                wo_hbm.at[:, pl.ds(c * CN, CN)], wo_vmem, wo_sem)
            cp.start()
            cp.wait()
            wo_bf = wo_vmem[...].astype(jnp.bfloat16)

            other = jnp.dot(o_ref[pl.ds(other_start, S_HALF), :], wo_bf,
                            preferred_element_type=jnp.float32)
            send_buf[...] = other.astype(jnp.bfloat16)

            @pl.when(c > 0)
            def _():
                pl.semaphore_wait(credit_sem, 1)

            rdma = pltpu.make_async_remote_copy(
                src_ref=send_buf, dst_ref=recv_buf,
                send_sem=send_sem, recv_sem=recv_sem,
                device_id=peer, device_id_type=pl.DeviceIdType.MESH)
            rdma.start()

            own = jnp.dot(o_ref[pl.ds(own_start, S_HALF), :], wo_bf,
                          preferred_element_type=jnp.float32)
            rdma.wait()

            out_vmem[...] = own + recv_buf[...].astype(jnp.float32)

            @pl.when(c < NCHUNK - 1)
            def _():
                pl.semaphore_signal(credit_sem, inc=1, device_id=peer,
                                    device_id_type=pl.DeviceIdType.MESH)

            ocp = pltpu.make_async_copy(
                out_vmem, out_hbm.at[0, :, pl.ds(c * CN, CN)], out_sem)
            ocp.start()
            ocp.wait()
            return carry

        lax.fori_loop(0, NCHUNK, chunk_step, 0)

    return pl.pallas_call(
        body,
        out_shape=jax.ShapeDtypeStruct((1, S_HALF, N), jnp.float32),
        in_specs=[
            pl.BlockSpec(memory_space=pltpu.VMEM),
            pl.BlockSpec(memory_space=pl.ANY),
        ],
        out_specs=pl.BlockSpec(memory_space=pl.ANY),
        scratch_shapes=[
            pltpu.VMEM((K, CN), jnp.float32),
            pltpu.VMEM((S_HALF, CN), jnp.bfloat16),
            pltpu.VMEM((S_HALF, CN), jnp.bfloat16),
            pltpu.VMEM((S_HALF, CN), jnp.float32),
            pltpu.SemaphoreType.DMA,
            pltpu.SemaphoreType.DMA,
            pltpu.SemaphoreType.DMA,
            pltpu.SemaphoreType.DMA,
            pltpu.SemaphoreType.REGULAR,
        ],
        compiler_params=pltpu.CompilerParams(collective_id=0),
    )(O2, Wo)


# device time: 328727 ns/iter; 1.4970x vs baseline; 1.4970x over previous
import jax
import jax.numpy as jnp
from jax import lax
from jax.experimental import pallas as pl
from jax.experimental.pallas import tpu as pltpu

S = 2048
K = 4096
N = 8192
S_HALF = 1024
CN = 512
NC_LOCAL = 8
PAIRS = NC_LOCAL // 2


def kernel(O, Wo):
    O2 = O.reshape(S, K).astype(jnp.bfloat16)
    Wo_bf = Wo.astype(jnp.bfloat16)

    def body(o_ref, wo_hbm, out_hbm,
             wo_vmem, xsend, xrecv, ysend, yrecv, sum_vmem, yout_vmem,
             wo_sems, xsend_sem, xrecv_sem, ysend_sems, yrecv_sems,
             out_sem, xcredit, ycredit):
        my_x = lax.axis_index("x")
        my_y = lax.axis_index("y")
        xpeer = (1 - my_x, my_y)
        ypartner = (my_x, 1 - my_y)

        barrier = pltpu.get_barrier_semaphore()
        for nbr in (xpeer, ypartner):
            pl.semaphore_signal(barrier, inc=1, device_id=nbr,
                                device_id_type=pl.DeviceIdType.MESH)
        pl.semaphore_wait(barrier, 2)

        own_start = my_x * S_HALF
        other_start = (1 - my_x) * S_HALF
        base = my_y * NC_LOCAL
        pbase = (1 - my_y) * NC_LOCAL

        def wo_copy(chunk, slot):
            return pltpu.make_async_copy(
                wo_hbm.at[:, pl.ds(chunk * CN, CN)],
                wo_vmem.at[slot], wo_sems.at[slot])

        def x_rdma():
            return pltpu.make_async_remote_copy(
                src_ref=xsend, dst_ref=xrecv,
                send_sem=xsend_sem, recv_sem=xrecv_sem,
                device_id=xpeer, device_id_type=pl.DeviceIdType.MESH)

        def y_rdma(slot):
            return pltpu.make_async_remote_copy(
                src_ref=ysend.at[slot], dst_ref=yrecv.at[slot],
                send_sem=ysend_sems.at[slot], recv_sem=yrecv_sems.at[slot],
                device_id=ypartner, device_id_type=pl.DeviceIdType.MESH)

        def out_copy(src, chunk):
            return pltpu.make_async_copy(
                src, out_hbm.at[0, :, pl.ds(chunk * CN, CN)], out_sem)

        for s in range(2):
            wo_copy(base + s, s).start()

        def pair_step(j, carry):
            for k in range(2):
                i = 2 * j + k
                slot = k
                not_first = (j > 0) if k == 0 else None
                steady = j > 0

                wo_copy(base + i, slot).wait()
                wo_bf = wo_vmem[slot]

                other = jnp.dot(o_ref[pl.ds(other_start, S_HALF), :], wo_bf,
                                preferred_element_type=jnp.float32)
                xsend[...] = other.astype(jnp.bfloat16)

                def x_gated():
                    pl.semaphore_wait(xcredit, 1)
                if k == 0:
                    pl.when(not_first)(x_gated)
                else:
                    x_gated()
                x_rdma().start()

                own = jnp.dot(o_ref[pl.ds(own_start, S_HALF), :], wo_bf,
                              preferred_element_type=jnp.float32)

                def consume_prev():
                    pslot = 1 - k
                    y_rdma(pslot).wait_recv()
                    yout_vmem[...] = yrecv[pslot].astype(jnp.float32)
                    if k == 1:
                        @pl.when(j < PAIRS - 1)
                        def _():
                            pl.semaphore_signal(
                                ycredit, inc=1, device_id=ypartner,
                                device_id_type=pl.DeviceIdType.MESH)
                    else:
                        pl.semaphore_signal(
                            ycredit, inc=1, device_id=ypartner,
                            device_id_type=pl.DeviceIdType.MESH)
                    cc = pbase + i - 1
                    ocp = out_copy(yout_vmem, cc)
                    ocp.start()
                    ocp.wait()
                if k == 0:
                    pl.when(not_first)(consume_prev)
                else:
                    consume_prev()

                x_rdma().wait()
                ssum = own + xrecv[...].astype(jnp.float32)
                sum_vmem[...] = ssum

                @pl.when(j < PAIRS - 1)
                def _():
                    wo_copy(base + i + 2, slot).start()

                def x_credit():
                    pl.semaphore_signal(xcredit, inc=1, device_id=xpeer,
                                        device_id_type=pl.DeviceIdType.MESH)
                if k == 0:
                    x_credit()
                else:
                    pl.when(j < PAIRS - 1)(x_credit)

                @pl.when(steady)
                def _():
                    y_rdma(slot).wait_send()
                    pl.semaphore_wait(ycredit, 1)
                ysend[slot] = ssum.astype(jnp.bfloat16)
                y_rdma(slot).start()

                ocp = out_copy(sum_vmem, base + i)
                ocp.start()
                ocp.wait()
            return carry

        lax.fori_loop(0, PAIRS, pair_step, 0)

        y_rdma(1).wait_recv()
        yout_vmem[...] = yrecv[1].astype(jnp.float32)
        ocp = out_copy(yout_vmem, pbase + NC_LOCAL - 1)
        ocp.start()
        ocp.wait()
        y_rdma(0).wait_send()
        y_rdma(1).wait_send()

    return pl.pallas_call(
        body,
        out_shape=jax.ShapeDtypeStruct((1, S_HALF, N), jnp.float32),
        in_specs=[
            pl.BlockSpec(memory_space=pltpu.VMEM),
            pl.BlockSpec(memory_space=pl.ANY),
        ],
        out_specs=pl.BlockSpec(memory_space=pl.ANY),
        scratch_shapes=[
            pltpu.VMEM((2, K, CN), jnp.bfloat16),
            pltpu.VMEM((S_HALF, CN), jnp.bfloat16),
            pltpu.VMEM((S_HALF, CN), jnp.bfloat16),
            pltpu.VMEM((2, S_HALF, CN), jnp.bfloat16),
            pltpu.VMEM((2, S_HALF, CN), jnp.bfloat16),
            pltpu.VMEM((S_HALF, CN), jnp.float32),
            pltpu.VMEM((S_HALF, CN), jnp.float32),
            pltpu.SemaphoreType.DMA((2,)),
            pltpu.SemaphoreType.DMA,
            pltpu.SemaphoreType.DMA,
            pltpu.SemaphoreType.DMA((2,)),
            pltpu.SemaphoreType.DMA((2,)),
            pltpu.SemaphoreType.DMA,
            pltpu.SemaphoreType.REGULAR,
            pltpu.SemaphoreType.REGULAR,
        ],
        compiler_params=pltpu.CompilerParams(collective_id=0),
    )(O2, Wo_bf)


# device time: 264606 ns/iter; 1.8598x vs baseline; 1.2423x over previous
import jax
import jax.numpy as jnp
from jax import lax
from jax.experimental import pallas as pl
from jax.experimental.pallas import tpu as pltpu

S = 2048
K = 4096
N = 8192
S_HALF = 1024
CN = 512
NC_LOCAL = 8
PAIRS = NC_LOCAL // 2


def kernel(O, Wo):
    O2 = O.reshape(S, K).astype(jnp.bfloat16)

    def body(o_ref, wo_hbm, out_hbm,
             wo_vmem, xsend, xrecv, ysend, yrecv, sum_vmem, yout_vmem,
             wo_sems, xsend_sem, xrecv_sem, ysend_sems, yrecv_sems,
             out_sem, xcredit, ycredit):
        my_x = lax.axis_index("x")
        my_y = lax.axis_index("y")
        xpeer = (1 - my_x, my_y)
        ypartner = (my_x, 1 - my_y)

        barrier = pltpu.get_barrier_semaphore()
        for nbr in (xpeer, ypartner):
            pl.semaphore_signal(barrier, inc=1, device_id=nbr,
                                device_id_type=pl.DeviceIdType.MESH)
        pl.semaphore_wait(barrier, 2)

        own_start = my_x * S_HALF
        other_start = (1 - my_x) * S_HALF
        base = my_y * NC_LOCAL
        pbase = (1 - my_y) * NC_LOCAL

        def wo_copy(chunk, slot):
            return pltpu.make_async_copy(
                wo_hbm.at[:, pl.ds(chunk * CN, CN)],
                wo_vmem.at[slot], wo_sems.at[slot])

        def x_rdma():
            return pltpu.make_async_remote_copy(
                src_ref=xsend, dst_ref=xrecv,
                send_sem=xsend_sem, recv_sem=xrecv_sem,
                device_id=xpeer, device_id_type=pl.DeviceIdType.MESH)

        def y_rdma(slot):
            return pltpu.make_async_remote_copy(
                src_ref=ysend.at[slot], dst_ref=yrecv.at[slot],
                send_sem=ysend_sems.at[slot], recv_sem=yrecv_sems.at[slot],
                device_id=ypartner, device_id_type=pl.DeviceIdType.MESH)

        def out_copy(src, chunk):
            return pltpu.make_async_copy(
                src, out_hbm.at[0, :, pl.ds(chunk * CN, CN)], out_sem)

        for s in range(2):
            wo_copy(base + s, s).start()

        def pair_step(j, carry):
            for k in range(2):
                i = 2 * j + k
                slot = k
                not_first = (j > 0) if k == 0 else None
                steady = j > 0

                wo_copy(base + i, slot).wait()
                wo_bf = wo_vmem[slot].astype(jnp.bfloat16)

                other = jnp.dot(o_ref[pl.ds(other_start, S_HALF), :], wo_bf,
                                preferred_element_type=jnp.float32)
                xsend[...] = other.astype(jnp.bfloat16)

                def x_gated():
                    pl.semaphore_wait(xcredit, 1)
                if k == 0:
                    pl.when(not_first)(x_gated)
                else:
                    x_gated()
                x_rdma().start()

                own = jnp.dot(o_ref[pl.ds(own_start, S_HALF), :], wo_bf,
                              preferred_element_type=jnp.float32)

                def consume_prev():
                    pslot = 1 - k
                    y_rdma(pslot).wait_recv()
                    yout_vmem[...] = yrecv[pslot].astype(jnp.float32)
                    if k == 1:
                        @pl.when(j < PAIRS - 1)
                        def _():
                            pl.semaphore_signal(
                                ycredit, inc=1, device_id=ypartner,
                                device_id_type=pl.DeviceIdType.MESH)
                    else:
                        pl.semaphore_signal(
                            ycredit, inc=1, device_id=ypartner,
                            device_id_type=pl.DeviceIdType.MESH)
                    cc = pbase + i - 1
                    ocp = out_copy(yout_vmem, cc)
                    ocp.start()
                    ocp.wait()
                if k == 0:
                    pl.when(not_first)(consume_prev)
                else:
                    consume_prev()

                x_rdma().wait()
                ssum = own + xrecv[...].astype(jnp.float32)
                sum_vmem[...] = ssum

                @pl.when(j < PAIRS - 1)
                def _():
                    wo_copy(base + i + 2, slot).start()

                def x_credit():
                    pl.semaphore_signal(xcredit, inc=1, device_id=xpeer,
                                        device_id_type=pl.DeviceIdType.MESH)
                if k == 0:
                    x_credit()
                else:
                    pl.when(j < PAIRS - 1)(x_credit)

                @pl.when(steady)
                def _():
                    y_rdma(slot).wait_send()
                    pl.semaphore_wait(ycredit, 1)
                ysend[slot] = ssum.astype(jnp.bfloat16)
                y_rdma(slot).start()

                ocp = out_copy(sum_vmem, base + i)
                ocp.start()
                ocp.wait()
            return carry

        lax.fori_loop(0, PAIRS, pair_step, 0)

        y_rdma(1).wait_recv()
        yout_vmem[...] = yrecv[1].astype(jnp.float32)
        ocp = out_copy(yout_vmem, pbase + NC_LOCAL - 1)
        ocp.start()
        ocp.wait()
        y_rdma(0).wait_send()
        y_rdma(1).wait_send()

    return pl.pallas_call(
        body,
        out_shape=jax.ShapeDtypeStruct((1, S_HALF, N), jnp.float32),
        in_specs=[
            pl.BlockSpec(memory_space=pltpu.VMEM),
            pl.BlockSpec(memory_space=pl.ANY),
        ],
        out_specs=pl.BlockSpec(memory_space=pl.ANY),
        scratch_shapes=[
            pltpu.VMEM((2, K, CN), jnp.float32),
            pltpu.VMEM((S_HALF, CN), jnp.bfloat16),
            pltpu.VMEM((S_HALF, CN), jnp.bfloat16),
            pltpu.VMEM((2, S_HALF, CN), jnp.bfloat16),
            pltpu.VMEM((2, S_HALF, CN), jnp.bfloat16),
            pltpu.VMEM((S_HALF, CN), jnp.float32),
            pltpu.VMEM((S_HALF, CN), jnp.float32),
            pltpu.SemaphoreType.DMA((2,)),
            pltpu.SemaphoreType.DMA,
            pltpu.SemaphoreType.DMA,
            pltpu.SemaphoreType.DMA((2,)),
            pltpu.SemaphoreType.DMA((2,)),
            pltpu.SemaphoreType.DMA,
            pltpu.SemaphoreType.REGULAR,
            pltpu.SemaphoreType.REGULAR,
        ],
        compiler_params=pltpu.CompilerParams(
            collective_id=0, vmem_limit_bytes=60 * 1024 * 1024),
    )(O2, Wo)


# device time: 222034 ns/iter; 2.2164x vs baseline; 1.1917x over previous
import jax
import jax.numpy as jnp
from jax import lax
from jax.experimental import pallas as pl
from jax.experimental.pallas import tpu as pltpu

S = 2048
K = 4096
N = 8192
S_HALF = 1024
CN = 512
NC_LOCAL = 8
PAIRS = NC_LOCAL // 2


def kernel(O, Wo):
    O2 = O.reshape(S, K).astype(jnp.bfloat16)

    def body(o_ref, wo_hbm, out_hbm,
             wo_vmem, xsend, xrecv, ysend, yrecv, own_stage,
             wo_sems, xsend_sems, xrecv_sems, ysend_sems, yrecv_sems,
             out_sem):
        my_x = lax.axis_index("x")
        my_y = lax.axis_index("y")
        xpeer = (1 - my_x, my_y)
        ypartner = (my_x, 1 - my_y)

        barrier = pltpu.get_barrier_semaphore()
        for nbr in (xpeer, ypartner):
            pl.semaphore_signal(barrier, inc=1, device_id=nbr,
                                device_id_type=pl.DeviceIdType.MESH)
        pl.semaphore_wait(barrier, 2)

        own_start = my_x * S_HALF
        other_start = (1 - my_x) * S_HALF
        base = my_y * NC_LOCAL
        pbase = (1 - my_y) * NC_LOCAL

        def wo_copy(chunk, slot):
            return pltpu.make_async_copy(
                wo_hbm.at[:, pl.ds(chunk * CN, CN)],
                wo_vmem.at[slot], wo_sems.at[slot])

        def x_rdma(slot):
            return pltpu.make_async_remote_copy(
                src_ref=xsend.at[slot], dst_ref=xrecv.at[slot],
                send_sem=xsend_sems.at[slot], recv_sem=xrecv_sems.at[slot],
                device_id=xpeer, device_id_type=pl.DeviceIdType.MESH)

        def y_rdma(slot):
            return pltpu.make_async_remote_copy(
                src_ref=ysend.at[slot], dst_ref=yrecv.at[slot],
                send_sem=ysend_sems.at[slot], recv_sem=yrecv_sems.at[slot],
                device_id=ypartner, device_id_type=pl.DeviceIdType.MESH)

        def out_copy(src, chunk):
            return pltpu.make_async_copy(
                src, out_hbm.at[0, :, pl.ds(chunk * CN, CN)], out_sem)

        def consume_x(i, p, gate_ysend):
            x_rdma(p).wait_recv()
            ssum = own_stage[p] + xrecv[p].astype(jnp.float32)
            if gate_ysend:
                y_rdma(p).wait_send()
            ysend[p] = ssum.astype(jnp.bfloat16)
            y_rdma(p).start()
            ocp = out_copy(ysend.at[p], base + i - 1)
            ocp.start()
            ocp.wait()

        def consume_y(i, s):
            y_rdma(s).wait_recv()
            ocp = out_copy(yrecv.at[s], pbase + i - 2)
            ocp.start()
            ocp.wait()

        for s in range(2):
            wo_copy(base + s, s).start()

        def pair_step(j, carry):
            for k in range(2):
                i = 2 * j + k
                s = k
                p = 1 - k

                wo_copy(base + i, s).wait()
                wo_bf = wo_vmem[s].astype(jnp.bfloat16)

                if k == 0:
                    pl.when(j > 1)(lambda: consume_x(i, p, True))
                    pl.when(j == 1)(lambda: consume_x(i, p, False))
                else:
                    pl.when(j > 0)(lambda: consume_x(i, p, True))
                    pl.when(j == 0)(lambda: consume_x(i, p, False))
                pl.when(j > 0)(lambda: consume_y(i, s))

                @pl.when(j > 0)
                def _():
                    x_rdma(s).wait_send()
                other = jnp.dot(o_ref[pl.ds(other_start, S_HALF), :], wo_bf,
                                preferred_element_type=jnp.float32)
                xsend[s] = other.astype(jnp.bfloat16)
                x_rdma(s).start()

                own = jnp.dot(o_ref[pl.ds(own_start, S_HALF), :], wo_bf,
                              preferred_element_type=jnp.float32)
                own_stage[s] = own

                @pl.when(j < PAIRS - 1)
                def _():
                    wo_copy(base + i + 2, s).start()
            return carry

        lax.fori_loop(0, PAIRS, pair_step, 0)

        consume_x(NC_LOCAL, 1, gate_ysend=True)
        consume_y(NC_LOCAL, 0)
        consume_y(NC_LOCAL + 1, 1)
        x_rdma(0).wait_send()
        x_rdma(1).wait_send()
        y_rdma(0).wait_send()
        y_rdma(1).wait_send()

    return pl.pallas_call(
        body,
        out_shape=jax.ShapeDtypeStruct((1, S_HALF, N), jnp.bfloat16),
        in_specs=[
            pl.BlockSpec(memory_space=pltpu.VMEM),
            pl.BlockSpec(memory_space=pl.ANY),
        ],
        out_specs=pl.BlockSpec(memory_space=pl.ANY),
        scratch_shapes=[
            pltpu.VMEM((2, K, CN), jnp.float32),
            pltpu.VMEM((2, S_HALF, CN), jnp.bfloat16),
            pltpu.VMEM((2, S_HALF, CN), jnp.bfloat16),
            pltpu.VMEM((2, S_HALF, CN), jnp.bfloat16),
            pltpu.VMEM((2, S_HALF, CN), jnp.bfloat16),
            pltpu.VMEM((2, S_HALF, CN), jnp.float32),
            pltpu.SemaphoreType.DMA((2,)),
            pltpu.SemaphoreType.DMA((2,)),
            pltpu.SemaphoreType.DMA((2,)),
            pltpu.SemaphoreType.DMA((2,)),
            pltpu.SemaphoreType.DMA((2,)),
            pltpu.SemaphoreType.DMA,
        ],
        compiler_params=pltpu.CompilerParams(
            collective_id=0, vmem_limit_bytes=60 * 1024 * 1024),
    )(O2, Wo)


# device time: 168494 ns/iter; 2.9207x vs baseline; 1.3178x over previous
import jax
import jax.numpy as jnp
from jax import lax
from jax.experimental import pallas as pl
from jax.experimental.pallas import tpu as pltpu

S = 2048
K = 4096
N = 8192
S_HALF = 1024
CN = 512
NC_LOCAL = 8
UNROLL = 4
STEPS = NC_LOCAL // UNROLL


def kernel(O, Wo):
    O2 = O.reshape(S, K).astype(jnp.bfloat16)

    def body(o_ref, wo_hbm, out_hbm,
             wo_vmem, xsend, xrecv, ysend, yrecv, own_stage,
             wo_sems, xsend_sems, xrecv_sems, ysend_sems, yrecv_sems,
             out_sem):
        my_x = lax.axis_index("x")
        my_y = lax.axis_index("y")
        xpeer = (1 - my_x, my_y)
        ypartner = (my_x, 1 - my_y)

        barrier = pltpu.get_barrier_semaphore()
        for nbr in (xpeer, ypartner):
            pl.semaphore_signal(barrier, inc=1, device_id=nbr,
                                device_id_type=pl.DeviceIdType.MESH)
        pl.semaphore_wait(barrier, 2)

        own_start = my_x * S_HALF
        other_start = (1 - my_x) * S_HALF
        base = my_y * NC_LOCAL
        pbase = (1 - my_y) * NC_LOCAL

        def wo_copy(chunk, slot):
            return pltpu.make_async_copy(
                wo_hbm.at[:, pl.ds(chunk * CN, CN)],
                wo_vmem.at[slot], wo_sems.at[slot])

        def x_desc(sx, rx):
            return pltpu.make_async_remote_copy(
                src_ref=xsend.at[sx], dst_ref=xrecv.at[rx],
                send_sem=xsend_sems.at[sx], recv_sem=xrecv_sems.at[rx],
                device_id=xpeer, device_id_type=pl.DeviceIdType.MESH)

        def y_desc(slot):
            return pltpu.make_async_remote_copy(
                src_ref=ysend.at[slot], dst_ref=yrecv.at[slot],
                send_sem=ysend_sems.at[slot], recv_sem=yrecv_sems.at[slot],
                device_id=ypartner, device_id_type=pl.DeviceIdType.MESH)

        def out_copy(src, chunk):
            return pltpu.make_async_copy(
                src, out_hbm.at[0, :, pl.ds(chunk * CN, CN)], out_sem)

        def consume_x(chunk, rx, os_, gate_ysend):
            x_desc(rx % 2, rx).wait_recv()
            ssum = own_stage[os_] + xrecv[rx].astype(jnp.float32)
            if gate_ysend:
                y_desc(rx).wait_send()
            ysend[rx] = ssum.astype(jnp.bfloat16)
            y_desc(rx).start()
            ocp = out_copy(ysend.at[rx], chunk)
            ocp.start()
            ocp.wait()

        def consume_y(chunk, slot):
            y_desc(slot).wait_recv()
            ocp = out_copy(yrecv.at[slot], chunk)
            ocp.start()
            ocp.wait()

        for s in range(2):
            wo_copy(base + s, s).start()

        def step_fn(j, carry):
            for k in range(4):
                i = 4 * j + k

                wo_copy(base + i, k % 2).wait()
                wo_bf = wo_vmem[k % 2].astype(jnp.bfloat16)

                cxc = base + i - 2
                rx = (k - 2) % 4
                if k < 2:
                    pl.when(j > 0)(
                        lambda: consume_x(cxc, rx, k % 2, False))
                else:
                    pl.when(j == 0)(
                        lambda: consume_x(cxc, rx, k % 2, False))
                    pl.when(j > 0)(
                        lambda: consume_x(cxc, rx, k % 2, True))
                cyc = pbase + i - 3
                ry = (k - 3) % 4
                if k < 3:
                    pl.when(j > 0)(lambda: consume_y(cyc, ry))
                else:
                    consume_y(cyc, ry)

                if k < 2:
                    pl.when(j > 0)(lambda: x_desc(k % 2, 0).wait_send())
                else:
                    x_desc(k % 2, 0).wait_send()
                other = jnp.dot(o_ref[pl.ds(other_start, S_HALF), :], wo_bf,
                                preferred_element_type=jnp.float32)
                xsend[k % 2] = other.astype(jnp.bfloat16)
                x_desc(k % 2, k).start()

                own = jnp.dot(o_ref[pl.ds(own_start, S_HALF), :], wo_bf,
                              preferred_element_type=jnp.float32)
                own_stage[k % 2] = own

                if k < 2:
                    wo_copy(base + i + 2, k % 2).start()
                else:
                    pl.when(j == 0)(
                        lambda: wo_copy(base + i + 2, k % 2).start())
            return carry

        lax.fori_loop(0, STEPS, step_fn, 0)

        consume_x(base + 6, 2, 0, True)
        consume_x(base + 7, 3, 1, True)
        consume_y(pbase + 5, 1)
        consume_y(pbase + 6, 2)
        consume_y(pbase + 7, 3)
        x_desc(0, 0).wait_send()
        x_desc(1, 1).wait_send()
        for m in range(4):
            y_desc(m).wait_send()

    return pl.pallas_call(
        body,
        out_shape=jax.ShapeDtypeStruct((1, S_HALF, N), jnp.bfloat16),
        in_specs=[
            pl.BlockSpec(memory_space=pltpu.VMEM),
            pl.BlockSpec(memory_space=pl.ANY),
        ],
        out_specs=pl.BlockSpec(memory_space=pl.ANY),
        scratch_shapes=[
            pltpu.VMEM((2, K, CN), jnp.float32),
            pltpu.VMEM((2, S_HALF, CN), jnp.bfloat16),
            pltpu.VMEM((4, S_HALF, CN), jnp.bfloat16),
            pltpu.VMEM((4, S_HALF, CN), jnp.bfloat16),
            pltpu.VMEM((4, S_HALF, CN), jnp.bfloat16),
            pltpu.VMEM((2, S_HALF, CN), jnp.float32),
            pltpu.SemaphoreType.DMA((2,)),
            pltpu.SemaphoreType.DMA((2,)),
            pltpu.SemaphoreType.DMA((4,)),
            pltpu.SemaphoreType.DMA((4,)),
            pltpu.SemaphoreType.DMA((4,)),
            pltpu.SemaphoreType.DMA,
        ],
        compiler_params=pltpu.CompilerParams(
            collective_id=0, vmem_limit_bytes=62 * 1024 * 1024),
    )(O2, Wo)
